# baseline (speedup 1.0000x reference)
import functools

import jax
import jax.numpy as jnp
from jax import lax
from jax.experimental import pallas as pl
from jax.experimental.pallas import tpu as pltpu
from jax.experimental.pallas import tpu_sc as plsc

N_TOKENS = 8192
N_CODES = 8192
DIM = 256
COMMIT = 0.25

TN = 512
TK = 512
NNT = N_TOKENS // TN
NKT = N_CODES // TK


def _argmin_body(x_ref, e_ref, x2_ref, e2_ref, idx_ref, loss_ref,
                 m_ref, im_ref, acc_ref):
    n = pl.program_id(0)
    k = pl.program_id(1)
    s = lax.dot_general(x_ref[...], e_ref[...], (((1,), (1,)), ((), ())),
                        preferred_element_type=jnp.float32)
    d = (x2_ref[...] + e2_ref[...]) - 2.0 * s
    m_tile = jnp.min(d, axis=1, keepdims=True)
    jglob = k * TK + lax.broadcasted_iota(jnp.int32, (TN, TK), 1)
    big = jnp.int32(2**31 - 1)
    i_tile = jnp.min(jnp.where(d == m_tile, jglob, big), axis=1, keepdims=True)

    @pl.when(k == 0)
    def _():
        m_ref[...] = m_tile
        im_ref[...] = i_tile

    @pl.when(k > 0)
    def _():
        better = m_tile < m_ref[...]
        im_ref[...] = jnp.where(better, i_tile, im_ref[...])
        m_ref[...] = jnp.where(better, m_tile, m_ref[...])

    @pl.when(k == NKT - 1)
    def _():
        idx_ref[...] = im_ref[...]
        prev = jnp.where(n == 0, jnp.float32(0.0), acc_ref[0])
        acc_ref[0] = prev + jnp.sum(m_ref[...])

    @pl.when((k == NKT - 1) & (n == NNT - 1))
    def _():
        loss_ref[0] = acc_ref[0] * (COMMIT / (N_TOKENS * DIM))


_argmin_call = pl.pallas_call(
    _argmin_body,
    grid=(NNT, NKT),
    in_specs=[
        pl.BlockSpec((TN, DIM), lambda n, k: (n, 0)),
        pl.BlockSpec((TK, DIM), lambda n, k: (k, 0)),
        pl.BlockSpec((TN, 1), lambda n, k: (n, 0)),
        pl.BlockSpec((1, TK), lambda n, k: (0, k)),
    ],
    out_specs=[
        pl.BlockSpec((TN, 1), lambda n, k: (n, 0)),
        pl.BlockSpec(memory_space=pltpu.SMEM),
    ],
    out_shape=[
        jax.ShapeDtypeStruct((N_TOKENS, 1), jnp.int32),
        jax.ShapeDtypeStruct((1,), jnp.float32),
    ],
    scratch_shapes=[
        pltpu.VMEM((TN, 1), jnp.float32),
        pltpu.VMEM((TN, 1), jnp.int32),
        pltpu.SMEM((1,), jnp.float32),
    ],
)


def _sc_gather(table, idx):
    mesh = plsc.VectorSubcoreMesh(core_axis_name="c", subcore_axis_name="s")
    nw = mesh.num_cores * mesh.num_subcores
    bpw = N_TOKENS // nw
    nch = bpw // 128
    idx3 = idx.reshape(nw, nch, 128)

    @functools.partial(
        pl.kernel,
        mesh=mesh,
        out_type=jax.ShapeDtypeStruct((N_TOKENS, DIM), jnp.float32),
        scratch_types=[
            pltpu.VMEM((nch, 128), jnp.int32),
            pltpu.VMEM((bpw, DIM), jnp.float32),
            pltpu.SemaphoreType.DMA,
        ],
    )
    def gather_kernel(table_hbm, idx_hbm, out_hbm, idx_v, rows_v, sem):
        wid = lax.axis_index("s") * mesh.num_cores + lax.axis_index("c")
        base = wid * bpw
        pltpu.sync_copy(idx_hbm.at[wid], idx_v)
        copies = [
            pltpu.async_copy(table_hbm.at[idx_v.at[j]],
                             rows_v.at[pl.ds(j * 128, 128)], sem)
            for j in range(nch)
        ]
        for c in copies:
            c.wait()
        pltpu.sync_copy(rows_v, out_hbm.at[pl.ds(base, bpw)])

    return gather_kernel(table, idx3)


def kernel(inputs, emb_weight):
    inputs = inputs.astype(jnp.float32)
    B, C, H, W = inputs.shape
    flat = jnp.transpose(inputs, (0, 2, 3, 1)).reshape(-1, DIM)
    x2 = jnp.sum(flat ** 2, axis=1, keepdims=True)
    e2 = jnp.sum(emb_weight.T ** 2, axis=0, keepdims=True)
    idx2d, loss1 = _argmin_call(flat, emb_weight, x2, e2)
    q = _sc_gather(emb_weight, idx2d.reshape(-1))
    quantized = jnp.transpose(q.reshape(B, H, W, C), (0, 3, 1, 2))
    quantized_st = inputs + (quantized - inputs)
    return loss1[0], quantized_st
